# Initial kernel scaffold; baseline (speedup 1.0000x reference)
import jax
import jax.numpy as jnp
from jax.experimental import pallas as pl


def kernel(x, edge_index, edge_weight, W, b, gamma, beta):
    raise NotImplementedError("write your pallas kernel here")


# speedup vs baseline: 1.7635x; 1.7635x over previous
import functools

import jax
import jax.numpy as jnp
from jax import lax
from jax.experimental import pallas as pl
from jax.experimental.pallas import tpu as pltpu
from jax.experimental.pallas import tpu_sc as plsc

_N = 10000
_NP = 10240
_E = 160000
_D = 256
_DH = 128
_CHUNK = 128
_NSUB = 16
_ROWS_PER_TILE = _NP // _NSUB
_NCHUNKS = _E // _CHUNK
_BASE_CHUNKS = _NCHUNKS // _NSUB
_EXTRA = _NCHUNKS % _NSUB
_CROWS = _NP // _DH


def _sc_body(x0, x1, srcr, dstr, wr, z128,
             out0, out1, outc,
             src_idx, dst_idx, w_buf, cix, rows, cnt_v, acc, cshared, sem):
    cid = lax.axis_index("c")
    sid = lax.axis_index("s")
    stripe = pl.ds(sid * _ROWS_PER_TILE, _ROWS_PER_TILE)
    col16 = lax.iota(jnp.int32, 16)

    pltpu.sync_copy(z128.at[stripe], acc.at[stripe])
    pltpu.sync_copy(z128.at[pl.ds(0, _CROWS)], cnt_v)

    @pl.when(jnp.logical_and(cid == 0, sid == 0))
    def _():
        pltpu.sync_copy(z128.at[pl.ds(0, _CROWS)], cshared)

    for g in range(_CROWS // 16):
        cix[pl.ds(g * 16, 16)] = col16 + g * 16

    plsc.subcore_barrier()

    def process_chunk(c):
        base = c * _CHUNK
        pltpu.sync_copy(srcr.at[pl.ds(base, _CHUNK)], src_idx)
        pltpu.sync_copy(dstr.at[pl.ds(base, _CHUNK)], dst_idx)
        pltpu.sync_copy(wr.at[pl.ds(base, _CHUNK)], w_buf)

        @pl.when(cid == 0)
        def _():
            pltpu.async_copy(x0.at[src_idx], rows, sem).wait()

        @pl.when(cid == 1)
        def _():
            pltpu.async_copy(x1.at[src_idx], rows, sem).wait()

        def mul_body(k, c2):
            kspl = jnp.broadcast_to(k, (16,))
            wspl = plsc.load_gather(w_buf, [kspl])
            for j in range(_DH // 16):
                cidx = col16 + (j * 16)
                v = plsc.load_gather(rows, [kspl, cidx])
                plsc.store_scatter(rows, [kspl, cidx], v * wspl)
            return c2

        lax.fori_loop(0, _CHUNK, mul_body, 0)

        pltpu.sync_copy(rows, acc.at[dst_idx], add=True)

        @pl.when(cid == 0)
        def _():
            def cnt_body(k, c2):
                kspl = jnp.broadcast_to(k, (16,))
                dspl = plsc.load_gather(dst_idx, [kspl])
                d1 = lax.shift_right_logical(dspl, 7)
                d2 = jnp.bitwise_and(dspl, 127)
                cv = plsc.load_gather(cnt_v, [d1, d2])
                plsc.store_scatter(cnt_v, [d1, d2], cv + 1.0)
                return c2

            lax.fori_loop(0, _CHUNK, cnt_body, 0)

    def chunk_body(i, carry):
        process_chunk(sid * _BASE_CHUNKS + i)
        return carry

    lax.fori_loop(0, _BASE_CHUNKS, chunk_body, 0)

    @pl.when(sid < _EXTRA)
    def _():
        process_chunk(_NSUB * _BASE_CHUNKS + sid)

    @pl.when(cid == 0)
    def _():
        pltpu.sync_copy(cnt_v, cshared.at[cix], add=True)

    plsc.subcore_barrier()

    @pl.when(cid == 0)
    def _():
        pltpu.sync_copy(acc.at[stripe], out0.at[stripe])

    @pl.when(cid == 1)
    def _():
        pltpu.sync_copy(acc.at[stripe], out1.at[stripe])

    @pl.when(jnp.logical_and(cid == 0, sid == 0))
    def _():
        pltpu.sync_copy(cshared, outc)


_sc_agg = functools.partial(
    pl.kernel,
    out_type=[
        jax.ShapeDtypeStruct((_NP, _DH), jnp.float32),
        jax.ShapeDtypeStruct((_NP, _DH), jnp.float32),
        jax.ShapeDtypeStruct((_CROWS, _DH), jnp.float32),
    ],
    mesh=plsc.VectorSubcoreMesh(core_axis_name="c", subcore_axis_name="s"),
    compiler_params=pltpu.CompilerParams(needs_layout_passes=False),
    scratch_types=[
        pltpu.VMEM((_CHUNK,), jnp.int32),
        pltpu.VMEM((_CHUNK,), jnp.int32),
        pltpu.VMEM((_CHUNK,), jnp.float32),
        pltpu.VMEM((_CROWS,), jnp.int32),
        pltpu.VMEM((_CHUNK, _DH), jnp.float32),
        pltpu.VMEM((_CROWS, _DH), jnp.float32),
        pltpu.VMEM_SHARED((_NP, _DH), jnp.float32),
        pltpu.VMEM_SHARED((_CROWS, _DH), jnp.float32),
        pltpu.SemaphoreType.DMA,
    ],
)(_sc_body)


_BLK = 1000
_NBLK = _N // _BLK


def _t1_body(s0, s1, cr, w_ref, b_ref, g_ref, bt_ref, hpre, stats, accum):
    i = pl.program_id(0)

    @pl.when(i == 0)
    def _():
        accum[...] = jnp.zeros_like(accum)

    s = jnp.concatenate([s0[...], s1[...]], axis=1)
    cntv = jnp.maximum(cr[...][:, :1], 1.0)
    h = s / cntv
    h = lax.dot_general(h, w_ref[...], (((1,), (1,)), ((), ())),
                        preferred_element_type=jnp.float32)
    h = h + b_ref[...]
    hpre[...] = h
    accum[0:1, :] += jnp.sum(h, axis=0, keepdims=True)
    accum[1:2, :] += jnp.sum(h * h, axis=0, keepdims=True)

    @pl.when(i == _NBLK - 1)
    def _():
        mean = accum[0:1, :] / _N
        var = accum[1:2, :] / _N - mean * mean
        scale = g_ref[...] * lax.rsqrt(var + 1e-5)
        shift = bt_ref[...] - mean * scale
        stats[...] = jnp.concatenate([scale, shift], axis=0)


def _t2_body(hpre, stats, out):
    h = hpre[...] * stats[0:1, :] + stats[1:2, :]
    out[...] = jnp.maximum(h, 0.0)


def kernel(x, edge_index, edge_weight, W, b, gamma, beta):
    x0 = x[:, :_DH]
    x1 = x[:, _DH:]
    src = edge_index[0].astype(jnp.int32)
    dst = edge_index[1].astype(jnp.int32)
    w = edge_weight.astype(jnp.float32)
    z128 = jnp.zeros((_NP, _DH), jnp.float32)

    out0, out1, outc = _sc_agg(x0, x1, src, dst, w, z128)
    cntb = jnp.broadcast_to(outc.reshape(_NP, 1)[:_N], (_N, _DH))
    out0, out1 = out0[:_N], out1[:_N]

    hpre, stats = pl.pallas_call(
        _t1_body,
        grid=(_NBLK,),
        in_specs=[
            pl.BlockSpec((_BLK, _DH), lambda i: (i, 0)),
            pl.BlockSpec((_BLK, _DH), lambda i: (i, 0)),
            pl.BlockSpec((_BLK, _DH), lambda i: (i, 0)),
            pl.BlockSpec((_D, _D), lambda i: (0, 0)),
            pl.BlockSpec((1, _D), lambda i: (0, 0)),
            pl.BlockSpec((1, _D), lambda i: (0, 0)),
            pl.BlockSpec((1, _D), lambda i: (0, 0)),
        ],
        out_specs=[
            pl.BlockSpec((_BLK, _D), lambda i: (i, 0)),
            pl.BlockSpec((2, _D), lambda i: (0, 0)),
        ],
        out_shape=[
            jax.ShapeDtypeStruct((_N, _D), jnp.float32),
            jax.ShapeDtypeStruct((2, _D), jnp.float32),
        ],
        scratch_shapes=[pltpu.VMEM((2, _D), jnp.float32)],
    )(out0, out1, cntb, W, b.reshape(1, _D), gamma.reshape(1, _D),
      beta.reshape(1, _D))

    h = pl.pallas_call(
        _t2_body,
        grid=(_NBLK,),
        in_specs=[
            pl.BlockSpec((_BLK, _D), lambda i: (i, 0)),
            pl.BlockSpec((2, _D), lambda i: (0, 0)),
        ],
        out_specs=pl.BlockSpec((_BLK, _D), lambda i: (i, 0)),
        out_shape=jax.ShapeDtypeStruct((_N, _D), jnp.float32),
    )(hpre, stats)
    return h


# speedup vs baseline: 4.1989x; 2.3811x over previous
import functools

import jax
import jax.numpy as jnp
from jax import lax
from jax.experimental import pallas as pl
from jax.experimental.pallas import tpu as pltpu
from jax.experimental.pallas import tpu_sc as plsc

_N = 10000
_NP = 10240
_E = 160000
_D = 256
_DH = 128
_CHUNK = 128
_NSUB = 16
_ROWS_PER_TILE = _NP // _NSUB
_NCHUNKS = _E // _CHUNK
_BASE_CHUNKS = _NCHUNKS // _NSUB
_EXTRA = _NCHUNKS % _NSUB
_CROWS = _NP // _DH


def _sc_body(x0, x1, srcr, dstr, wr, z128,
             out0, out1, outc0, outc1,
             src_idx0, dst_idx0, w_buf0, src_idx1, dst_idx1, w_buf1, cix,
             rows0, rows1, cnt_v, acc, cshared, gsem0, gsem1):
    cid = lax.axis_index("c")
    sid = lax.axis_index("s")
    stripe = pl.ds(sid * _ROWS_PER_TILE, _ROWS_PER_TILE)
    col16 = lax.iota(jnp.int32, 16)

    def load_idx(c, si, di, wb):
        base = c * _CHUNK
        pltpu.sync_copy(srcr.at[pl.ds(base, _CHUNK)], si)
        pltpu.sync_copy(dstr.at[pl.ds(base, _CHUNK)], di)
        pltpu.sync_copy(wr.at[pl.ds(base, _CHUNK)], wb)

    def start_gather(si, rows_b, sem):
        @pl.when(cid == 0)
        def _():
            pltpu.async_copy(x0.at[si], rows_b, sem)

        @pl.when(cid == 1)
        def _():
            pltpu.async_copy(x1.at[si], rows_b, sem)

    def wait_gather(si, rows_b, sem):
        pltpu.make_async_copy(x0.at[si], rows_b, sem).wait()

    def scale(rows_b, wb):
        @plsc.parallel_loop(0, _CHUNK, 1, unroll=4)
        def _(k):
            kspl = jnp.broadcast_to(k, (16,))
            wspl = plsc.load_gather(wb, [kspl])
            for j in range(_DH // 16):
                cidx = col16 + (j * 16)
                v = plsc.load_gather(rows_b, [kspl, cidx])
                plsc.store_scatter(rows_b, [kspl, cidx], v * wspl)

    def count(di):
        def cnt_body(k, c2):
            kspl = jnp.broadcast_to(k + cid * (_CHUNK // 2), (16,))
            dspl = plsc.load_gather(di, [kspl])
            d1 = lax.shift_right_logical(dspl, 7)
            d2 = jnp.bitwise_and(dspl, 127)
            cv = plsc.load_gather(cnt_v, [d1, d2])
            plsc.store_scatter(cnt_v, [d1, d2], cv + 1.0)
            return c2

        lax.fori_loop(0, _CHUNK // 2, cnt_body, 0)

    first = sid * _BASE_CHUNKS
    load_idx(first, src_idx0, dst_idx0, w_buf0)
    start_gather(src_idx0, rows0, gsem0)

    pltpu.sync_copy(z128.at[stripe], acc.at[stripe])
    pltpu.sync_copy(z128.at[pl.ds(0, _CROWS)], cnt_v)

    @pl.when(sid == 0)
    def _():
        pltpu.sync_copy(z128.at[pl.ds(0, _CROWS)], cshared)

    for g in range(_CROWS // 16):
        cix[pl.ds(g * 16, 16)] = col16 + g * 16

    plsc.subcore_barrier()

    def pair_body(i, carry):
        c0 = first + 2 * i
        load_idx(c0 + 1, src_idx1, dst_idx1, w_buf1)
        start_gather(src_idx1, rows1, gsem1)
        count(dst_idx0)
        wait_gather(src_idx0, rows0, gsem0)
        scale(rows0, w_buf0)
        pltpu.sync_copy(rows0, acc.at[dst_idx0], add=True)

        @pl.when(i < _BASE_CHUNKS // 2 - 1)
        def _():
            load_idx(c0 + 2, src_idx0, dst_idx0, w_buf0)
            start_gather(src_idx0, rows0, gsem0)

        count(dst_idx1)
        wait_gather(src_idx1, rows1, gsem1)
        scale(rows1, w_buf1)
        pltpu.sync_copy(rows1, acc.at[dst_idx1], add=True)
        return carry

    lax.fori_loop(0, _BASE_CHUNKS // 2, pair_body, 0)

    @pl.when(sid < _EXTRA)
    def _():
        c = _NSUB * _BASE_CHUNKS + sid
        load_idx(c, src_idx0, dst_idx0, w_buf0)
        start_gather(src_idx0, rows0, gsem0)
        count(dst_idx0)
        wait_gather(src_idx0, rows0, gsem0)
        scale(rows0, w_buf0)
        pltpu.sync_copy(rows0, acc.at[dst_idx0], add=True)

    pltpu.sync_copy(cnt_v, cshared.at[cix], add=True)

    plsc.subcore_barrier()

    @pl.when(cid == 0)
    def _():
        pltpu.sync_copy(acc.at[stripe], out0.at[stripe])

    @pl.when(cid == 1)
    def _():
        pltpu.sync_copy(acc.at[stripe], out1.at[stripe])

    @pl.when(jnp.logical_and(cid == 0, sid == 0))
    def _():
        pltpu.sync_copy(cshared, outc0)

    @pl.when(jnp.logical_and(cid == 1, sid == 0))
    def _():
        pltpu.sync_copy(cshared, outc1)


_sc_agg = functools.partial(
    pl.kernel,
    out_type=[
        jax.ShapeDtypeStruct((_NP, _DH), jnp.float32),
        jax.ShapeDtypeStruct((_NP, _DH), jnp.float32),
        jax.ShapeDtypeStruct((_CROWS, _DH), jnp.float32),
        jax.ShapeDtypeStruct((_CROWS, _DH), jnp.float32),
    ],
    mesh=plsc.VectorSubcoreMesh(core_axis_name="c", subcore_axis_name="s"),
    compiler_params=pltpu.CompilerParams(needs_layout_passes=False),
    scratch_types=[
        pltpu.VMEM((_CHUNK,), jnp.int32),
        pltpu.VMEM((_CHUNK,), jnp.int32),
        pltpu.VMEM((_CHUNK,), jnp.float32),
        pltpu.VMEM((_CHUNK,), jnp.int32),
        pltpu.VMEM((_CHUNK,), jnp.int32),
        pltpu.VMEM((_CHUNK,), jnp.float32),
        pltpu.VMEM((_CROWS,), jnp.int32),
        pltpu.VMEM((_CHUNK, _DH), jnp.float32),
        pltpu.VMEM((_CHUNK, _DH), jnp.float32),
        pltpu.VMEM((_CROWS, _DH), jnp.float32),
        pltpu.VMEM_SHARED((_NP, _DH), jnp.float32),
        pltpu.VMEM_SHARED((_CROWS, _DH), jnp.float32),
        pltpu.SemaphoreType.DMA,
        pltpu.SemaphoreType.DMA,
    ],
)(_sc_body)


_BLK = 1000
_NBLK = _N // _BLK


def _t1_body(s0, s1, cr0, cr1, w_ref, b_ref, g_ref, bt_ref, hpre, stats, accum):
    i = pl.program_id(0)

    @pl.when(i == 0)
    def _():
        accum[...] = jnp.zeros_like(accum)

    s = jnp.concatenate([s0[...], s1[...]], axis=1)
    cntv = jnp.maximum(cr0[...][:, :1] + cr1[...][:, :1], 1.0)
    h = s / cntv
    h = lax.dot_general(h, w_ref[...], (((1,), (1,)), ((), ())),
                        preferred_element_type=jnp.float32)
    h = h + b_ref[...]
    hpre[...] = h
    accum[0:1, :] += jnp.sum(h, axis=0, keepdims=True)
    accum[1:2, :] += jnp.sum(h * h, axis=0, keepdims=True)

    @pl.when(i == _NBLK - 1)
    def _():
        mean = accum[0:1, :] / _N
        var = accum[1:2, :] / _N - mean * mean
        scale = g_ref[...] * lax.rsqrt(var + 1e-5)
        shift = bt_ref[...] - mean * scale
        stats[...] = jnp.concatenate([scale, shift], axis=0)


def _t2_body(hpre, stats, out):
    h = hpre[...] * stats[0:1, :] + stats[1:2, :]
    out[...] = jnp.maximum(h, 0.0)


def kernel(x, edge_index, edge_weight, W, b, gamma, beta):
    x0 = x[:, :_DH]
    x1 = x[:, _DH:]
    src = edge_index[0].astype(jnp.int32)
    dst = edge_index[1].astype(jnp.int32)
    w = edge_weight.astype(jnp.float32)
    z128 = jnp.zeros((_NP, _DH), jnp.float32)

    out0, out1, outc0, outc1 = _sc_agg(x0, x1, src, dst, w, z128)
    cntb0 = jnp.broadcast_to(outc0.reshape(_NP, 1)[:_N], (_N, _DH))
    cntb1 = jnp.broadcast_to(outc1.reshape(_NP, 1)[:_N], (_N, _DH))
    out0, out1 = out0[:_N], out1[:_N]

    hpre, stats = pl.pallas_call(
        _t1_body,
        grid=(_NBLK,),
        in_specs=[
            pl.BlockSpec((_BLK, _DH), lambda i: (i, 0)),
            pl.BlockSpec((_BLK, _DH), lambda i: (i, 0)),
            pl.BlockSpec((_BLK, _DH), lambda i: (i, 0)),
            pl.BlockSpec((_BLK, _DH), lambda i: (i, 0)),
            pl.BlockSpec((_D, _D), lambda i: (0, 0)),
            pl.BlockSpec((1, _D), lambda i: (0, 0)),
            pl.BlockSpec((1, _D), lambda i: (0, 0)),
            pl.BlockSpec((1, _D), lambda i: (0, 0)),
        ],
        out_specs=[
            pl.BlockSpec((_BLK, _D), lambda i: (i, 0)),
            pl.BlockSpec((2, _D), lambda i: (0, 0)),
        ],
        out_shape=[
            jax.ShapeDtypeStruct((_N, _D), jnp.float32),
            jax.ShapeDtypeStruct((2, _D), jnp.float32),
        ],
        scratch_shapes=[pltpu.VMEM((2, _D), jnp.float32)],
    )(out0, out1, cntb0, cntb1, W, b.reshape(1, _D), gamma.reshape(1, _D),
      beta.reshape(1, _D))

    h = pl.pallas_call(
        _t2_body,
        grid=(_NBLK,),
        in_specs=[
            pl.BlockSpec((_BLK, _D), lambda i: (i, 0)),
            pl.BlockSpec((2, _D), lambda i: (0, 0)),
        ],
        out_specs=pl.BlockSpec((_BLK, _D), lambda i: (i, 0)),
        out_shape=jax.ShapeDtypeStruct((_N, _D), jnp.float32),
    )(hpre, stats)
    return h
